# baseline (speedup 1.0000x reference)
import jax
import jax.numpy as jnp
from jax import lax
from jax.experimental import pallas as pl
from jax.experimental.pallas import tpu as pltpu
from jax.experimental.pallas import tpu_sc as plsc

L = 16
NC = 2
NS = 16
NW = NC * NS

N_ROWS, N_COLS = 16384, 200
COLS_PER_W = N_ROWS // NW
CCHUNK = 128
N_CHUNKS = COLS_PER_W // CCHUNK
T01_SIZE = 100000
T12_SIZE = 1000
PK_WORDS = T01_SIZE // 4
W_PER_TILE = 1568
E_PER_TILE = 4 * W_PER_TILE
FUSE_VREGS = W_PER_TILE // L
HALF = N_COLS // 2


def _remap_body(x_hbm, t01_hbm, t12_hbm, out_hbm, pk_hbm,
                t01s_v, t12_v, pkl_v, pk_v, bufa, bufb,
                tsem, ia, ib, oa, ob):
    core = lax.axis_index("c")
    sid = lax.axis_index("s")
    wid = sid * NC + core
    cbase = wid * COLS_PER_W
    wb0 = jnp.where(sid == NS - 1, PK_WORDS - W_PER_TILE, sid * W_PER_TILE)

    t01_cp = pltpu.async_copy(
        t01_hbm.at[pl.ds(wb0 * 4, E_PER_TILE)], t01s_v, tsem)
    t12_cp = pltpu.async_copy(t12_hbm, t12_v, tsem)

    buf = [bufa, bufb]
    isem = [ia, ib]
    osem = [oa, ob]

    def in_copy(c, b):
        return pltpu.make_async_copy(
            x_hbm.at[:, pl.ds(cbase + c * CCHUNK, CCHUNK)], buf[b], isem[b])

    def out_copy(c, b):
        return pltpu.make_async_copy(
            buf[b], out_hbm.at[:, pl.ds(cbase + c * CCHUNK, CCHUNK)], osem[b])

    in_copy(0, 0).start()
    in_copy(1, 1).start()
    t01_cp.wait()
    t12_cp.wait()

    lanes4 = lax.iota(jnp.int32, L) * 4

    @plsc.parallel_loop(0, FUSE_VREGS, 1, unroll=2)
    def _fuse(j):
        ebase = j * (4 * L)
        w = None
        for m in range(4):
            v = plsc.load_gather(t01s_v, [lanes4 + (ebase + m)])
            f = plsc.load_gather(t12_v, [v])
            fm = f << (8 * m) if m else f
            w = fm if w is None else w | fm
        pkl_v[pl.ds(j * L, L)] = w

    pltpu.sync_copy(
        pkl_v, pk_hbm.at[pl.ds(core * PK_WORDS + wb0, W_PER_TILE)])
    plsc.subcore_barrier()
    pltpu.sync_copy(pk_hbm.at[pl.ds(core * PK_WORDS, PK_WORDS)], pk_v)

    def compute(b, r0, nrows):
        bc = buf[b]

        @plsc.parallel_loop(r0, r0 + nrows, 1, unroll=4)
        def _row(r):
            for c0 in range(0, CCHUNK, L):
                v = bc[r, pl.ds(c0, L)]
                w = plsc.load_gather(pk_v, [lax.shift_right_logical(v, 2)])
                sh = (v & 3) << 3
                bc[r, pl.ds(c0, L)] = lax.shift_right_logical(w, sh) & 0xFF

    in_copy(0, 0).wait()
    compute(0, 0, N_COLS)
    out_copy(0, 0).start()
    in_copy(1, 1).wait()
    compute(1, 0, HALF)
    out_copy(0, 0).wait()
    in_copy(2, 0).start()
    compute(1, HALF, N_COLS - HALF)
    out_copy(1, 1).start()
    in_copy(2, 0).wait()
    compute(0, 0, HALF)
    out_copy(1, 1).wait()
    in_copy(3, 1).start()
    compute(0, HALF, N_COLS - HALF)
    out_copy(2, 0).start()
    in_copy(3, 1).wait()
    compute(1, 0, N_COLS)
    out_copy(3, 1).start()

    out_copy(2, 0).wait()
    out_copy(3, 1).wait()


@jax.jit
def _remap(xt, table_01, table_12):
    mesh = plsc.VectorSubcoreMesh(core_axis_name="c", subcore_axis_name="s")
    out_t, _ = pl.kernel(
        _remap_body,
        mesh=mesh,
        out_type=(
            jax.ShapeDtypeStruct((N_COLS, N_ROWS), jnp.int32),
            jax.ShapeDtypeStruct((NC * PK_WORDS,), jnp.int32),
        ),
        scratch_types=[
            pltpu.VMEM((E_PER_TILE,), jnp.int32),
            pltpu.VMEM((T12_SIZE,), jnp.int32),
            pltpu.VMEM((W_PER_TILE,), jnp.int32),
            pltpu.VMEM((PK_WORDS,), jnp.int32),
            pltpu.VMEM((N_COLS, CCHUNK), jnp.int32),
            pltpu.VMEM((N_COLS, CCHUNK), jnp.int32),
            pltpu.SemaphoreType.DMA,
            pltpu.SemaphoreType.DMA,
            pltpu.SemaphoreType.DMA,
            pltpu.SemaphoreType.DMA,
            pltpu.SemaphoreType.DMA,
        ],
        compiler_params=pltpu.CompilerParams(needs_layout_passes=False),
    )(xt, table_01, table_12)
    return out_t


def kernel(x, table_01, table_12):
    return _remap(x.T, table_01, table_12).T


# speedup vs baseline: 1.0056x; 1.0056x over previous
import jax
import jax.numpy as jnp
from jax import lax
from jax.experimental import pallas as pl
from jax.experimental.pallas import tpu as pltpu
from jax.experimental.pallas import tpu_sc as plsc

L = 16
NC = 2
NS = 16
NW = NC * NS

N_ROWS, N_COLS = 16384, 200
COLS_PER_W = N_ROWS // NW
CCHUNK = 128
N_CHUNKS = COLS_PER_W // CCHUNK
T01_SIZE = 100000
T12_SIZE = 1000
PK_WORDS = T01_SIZE // 4
W_PER_TILE = 1568
E_PER_TILE = 4 * W_PER_TILE
FUSE_VREGS = W_PER_TILE // L
HALF = N_COLS // 2


def _remap_body(x_hbm, t01_hbm, t12_hbm, out_hbm, pk_hbm,
                t01s_v, t12_v, pkl_v, pk_v, bufa, bufb,
                tsem, ia, ib, oa, ob):
    core = lax.axis_index("c")
    sid = lax.axis_index("s")
    wid = sid * NC + core
    cbase = wid * COLS_PER_W
    wb0 = jnp.where(sid == NS - 1, PK_WORDS - W_PER_TILE, sid * W_PER_TILE)

    t01_cp = pltpu.async_copy(
        t01_hbm.at[pl.ds(wb0 * 4, E_PER_TILE)], t01s_v, tsem)
    t12_cp = pltpu.async_copy(t12_hbm, t12_v, tsem)

    buf = [bufa, bufb]
    isem = [ia, ib]
    osem = [oa, ob]

    def in_copy(c, b):
        return pltpu.make_async_copy(
            x_hbm.at[:, pl.ds(cbase + c * CCHUNK, CCHUNK)], buf[b], isem[b])

    def out_copy(c, b):
        return pltpu.make_async_copy(
            buf[b], out_hbm.at[:, pl.ds(cbase + c * CCHUNK, CCHUNK)], osem[b])

    in_copy(0, 0).start()
    in_copy(1, 1).start()
    t01_cp.wait()
    t12_cp.wait()

    lanes4 = lax.iota(jnp.int32, L) * 4

    @plsc.parallel_loop(0, FUSE_VREGS, 1, unroll=2)
    def _fuse(j):
        ebase = j * (4 * L)
        w = None
        for m in range(4):
            v = plsc.load_gather(t01s_v, [lanes4 + (ebase + m)])
            f = plsc.load_gather(t12_v, [v])
            fm = f << (8 * m) if m else f
            w = fm if w is None else w | fm
        pkl_v[pl.ds(j * L, L)] = w

    pltpu.sync_copy(
        pkl_v, pk_hbm.at[pl.ds(core * PK_WORDS + wb0, W_PER_TILE)])
    plsc.subcore_barrier()
    pltpu.sync_copy(pk_hbm.at[pl.ds(core * PK_WORDS, PK_WORDS)], pk_v)

    def compute(b, r0, nrows):
        bc = buf[b]

        @plsc.parallel_loop(r0, r0 + nrows, 1, unroll=2)
        def _row(r):
            for c0 in range(0, CCHUNK, L):
                v = bc[r, pl.ds(c0, L)]
                w = plsc.load_gather(pk_v, [lax.shift_right_logical(v, 2)])
                sh = (v & 3) << 3
                bc[r, pl.ds(c0, L)] = lax.shift_right_logical(w, sh) & 0xFF

    in_copy(0, 0).wait()
    compute(0, 0, N_COLS)
    out_copy(0, 0).start()
    in_copy(1, 1).wait()
    compute(1, 0, HALF)
    out_copy(0, 0).wait()
    in_copy(2, 0).start()
    compute(1, HALF, N_COLS - HALF)
    out_copy(1, 1).start()
    in_copy(2, 0).wait()
    compute(0, 0, HALF)
    out_copy(1, 1).wait()
    in_copy(3, 1).start()
    compute(0, HALF, N_COLS - HALF)
    out_copy(2, 0).start()
    in_copy(3, 1).wait()
    compute(1, 0, N_COLS)
    out_copy(3, 1).start()

    out_copy(2, 0).wait()
    out_copy(3, 1).wait()


@jax.jit
def _remap(xt, table_01, table_12):
    mesh = plsc.VectorSubcoreMesh(core_axis_name="c", subcore_axis_name="s")
    out_t, _ = pl.kernel(
        _remap_body,
        mesh=mesh,
        out_type=(
            jax.ShapeDtypeStruct((N_COLS, N_ROWS), jnp.int32),
            jax.ShapeDtypeStruct((NC * PK_WORDS,), jnp.int32),
        ),
        scratch_types=[
            pltpu.VMEM((E_PER_TILE,), jnp.int32),
            pltpu.VMEM((T12_SIZE,), jnp.int32),
            pltpu.VMEM((W_PER_TILE,), jnp.int32),
            pltpu.VMEM((PK_WORDS,), jnp.int32),
            pltpu.VMEM((N_COLS, CCHUNK), jnp.int32),
            pltpu.VMEM((N_COLS, CCHUNK), jnp.int32),
            pltpu.SemaphoreType.DMA,
            pltpu.SemaphoreType.DMA,
            pltpu.SemaphoreType.DMA,
            pltpu.SemaphoreType.DMA,
            pltpu.SemaphoreType.DMA,
        ],
        compiler_params=pltpu.CompilerParams(needs_layout_passes=False),
    )(xt, table_01, table_12)
    return out_t


def kernel(x, table_01, table_12):
    return _remap(x.T, table_01, table_12).T
